# baseline (speedup 1.0000x reference)
import jax
import jax.numpy as jnp
from jax import lax
from jax.experimental import pallas as pl
from jax.experimental.pallas import tpu as pltpu
from jax.experimental.pallas import tpu_sc as plsc

V = 32000
PAD = 0
N = 1600
NC = 2
NS = 16
L = 16
NW = NC * NS
PER_W = N // NW
SLOT = 64
CHUNKS = SLOT // L


def _loss_body(table_hbm, tgt_hbm, out_hbm, tgt_v, idx_v, red_v, buf_v, sem):
    cid = lax.axis_index("c")
    sid = lax.axis_index("s")
    wid = cid * NS + sid
    base = wid * PER_W

    w0 = pl.multiple_of(jnp.minimum((base // 8) * 8, N - SLOT), 8)
    off = base - w0
    pltpu.sync_copy(tgt_hbm.at[pl.ds(w0, SLOT)], tgt_v.at[pl.ds(0, SLOT)])

    lanes = lax.iota(jnp.int32, L)
    chunks = []
    for j in range(CHUNKS):
        k = j * L
        t = tgt_v[pl.ds(off + k, L)]
        chunks.append(t)
        valid = (k + lanes) < PER_W
        r = base + k + lanes
        p = ((r >> 3) * 2000) + ((t >> 7) << 3) + (r & 7)
        idx_v[pl.ds(k, L)] = jnp.where(valid, p, k + lanes)

    pltpu.async_copy(table_hbm.at[idx_v], buf_v, sem).wait()

    acc = jnp.zeros((L,), jnp.float32)
    for j in range(CHUNKS):
        k = j * L
        t = chunks[j]
        valid = (k + lanes) < PER_W
        rows = k + lanes
        lane = jnp.where(valid, t & 127, 0)
        vals = plsc.load_gather(buf_v, [rows, lane])
        acc = acc - jnp.where(valid & (t != PAD), vals, 0.0)

    red_v[...] = acc
    pltpu.sync_copy(red_v, out_hbm.at[pl.ds(wid * L, L)])


def kernel(outputs, targets):
    table = outputs.reshape(200, 8, 250, 128).transpose(0, 2, 1, 3)
    table = table.reshape(N * V // 128, 128)
    tgt = jnp.ravel(targets).astype(jnp.int32)

    mesh = plsc.VectorSubcoreMesh(
        core_axis_name="c", subcore_axis_name="s",
        num_cores=NC, num_subcores=NS)
    run = pl.kernel(
        _loss_body,
        out_type=jax.ShapeDtypeStruct((NW * L,), jnp.float32),
        mesh=mesh,
        compiler_params=pltpu.CompilerParams(use_tc_tiling_on_sc=True,
                                             needs_layout_passes=False),
        scratch_types=[
            pltpu.VMEM((SLOT + L,), jnp.int32),
            pltpu.VMEM((SLOT,), jnp.int32),
            pltpu.VMEM((L,), jnp.float32),
            pltpu.VMEM((SLOT, 128), jnp.float32),
            pltpu.SemaphoreType.DMA,
        ],
    )
    out = run(table, tgt)
    return jnp.sum(out)


# speedup vs baseline: 1.0103x; 1.0103x over previous
import jax
import jax.numpy as jnp
from jax import lax
from jax.experimental import pallas as pl
from jax.experimental.pallas import tpu as pltpu
from jax.experimental.pallas import tpu_sc as plsc

V = 32000
PAD = 0
N = 1600
NC = 2
NS = 16
L = 16
NW = NC * NS
PER_W = N // NW
SLOT = 64
CHUNKS = SLOT // L


def _loss_body(table_hbm, tgt_hbm, out_hbm, tgt_v, idx_v, red_v, buf_v, sem):
    cid = lax.axis_index("c")
    sid = lax.axis_index("s")
    wid = cid * NS + sid
    base = wid * PER_W

    pltpu.sync_copy(tgt_hbm.at[pl.ds(wid * SLOT, SLOT)], tgt_v)

    lanes = lax.iota(jnp.int32, L)
    for j in range(CHUNKS):
        k = j * L
        t = tgt_v[pl.ds(k, L)]
        valid = (k + lanes) < PER_W
        r = base + k + lanes
        p = ((r >> 3) * 2000) + ((t >> 7) << 3) + (r & 7)
        idx_v[pl.ds(k, L)] = jnp.where(valid, p, k + lanes)

    pltpu.async_copy(table_hbm.at[idx_v], buf_v, sem).wait()

    acc = jnp.zeros((L,), jnp.float32)
    for j in range(CHUNKS):
        k = j * L
        t = tgt_v[pl.ds(k, L)]
        valid = (k + lanes) < PER_W
        rows = k + lanes
        lane = jnp.where(valid, t & 127, 0)
        vals = plsc.load_gather(buf_v, [rows, lane])
        acc = acc - jnp.where(valid & (t != PAD), vals, 0.0)

    red_v[...] = acc
    pltpu.sync_copy(red_v, out_hbm.at[pl.ds(wid * L, L)])


def kernel(outputs, targets):
    table = outputs.reshape(200, 8, 250, 128).transpose(0, 2, 1, 3)
    table = table.reshape(N * V // 128, 128)
    tgt = jnp.ravel(targets).astype(jnp.int32)
    tgt = jnp.pad(tgt.reshape(NW, PER_W),
                  ((0, 0), (0, SLOT - PER_W))).reshape(-1)

    mesh = plsc.VectorSubcoreMesh(
        core_axis_name="c", subcore_axis_name="s",
        num_cores=NC, num_subcores=NS)
    run = pl.kernel(
        _loss_body,
        out_type=jax.ShapeDtypeStruct((NW * L,), jnp.float32),
        mesh=mesh,
        compiler_params=pltpu.CompilerParams(use_tc_tiling_on_sc=True,
                                             needs_layout_passes=False),
        scratch_types=[
            pltpu.VMEM((SLOT,), jnp.int32),
            pltpu.VMEM((SLOT,), jnp.int32),
            pltpu.VMEM((L,), jnp.float32),
            pltpu.VMEM((SLOT, 128), jnp.float32),
            pltpu.SemaphoreType.DMA,
        ],
    )
    out = run(table, tgt)
    return jnp.sum(out)
